# baseline (speedup 1.0000x reference)
import functools

import jax
import jax.numpy as jnp
from jax import lax
from jax.experimental import pallas as pl
from jax.experimental.pallas import tpu as pltpu
from jax.experimental.pallas import tpu_sc as plsc

N_NODES = 10000
N_EDGES = 320000
D_FEAT = 128
N_HIDDEN = 64
N_CLASSES = 16

NUM_CORES = 2
NUM_SUBCORES = 16
NUM_TILES = NUM_CORES * NUM_SUBCORES
N_PAD = 10240
REAL_EPT = N_EDGES // NUM_TILES
SL_PT = N_PAD // NUM_TILES
CHUNK = 80
EPT = REAL_EPT
NCHUNK = EPT // CHUNK
ROWS_PER_TILE = N_PAD // NUM_SUBCORES
RZ = 128
NB = 5
DEG_W = 16
PK = N_PAD * N_CLASSES // 128

_MESH = plsc.VectorSubcoreMesh(core_axis_name="c", subcore_axis_name="s")


def _zero_fill(buf, nrows, width):
    z = jnp.zeros((16,), jnp.float32)
    for r in range(nrows):
        for c in range(width // 16):
            buf[r, pl.ds(c * 16, 16)] = z


def _make_push(width, out_width):
    @functools.partial(
        pl.kernel,
        out_type=jax.ShapeDtypeStruct((NUM_CORES, N_PAD, out_width), jnp.float32),
        mesh=_MESH,
        compiler_params=pltpu.CompilerParams(use_tc_tiling_on_sc=False),
        scratch_types=[
            pltpu.VMEM((EPT,), jnp.int32),
            pltpu.VMEM((NCHUNK, CHUNK), jnp.int32),
            pltpu.VMEM((2, NB * CHUNK, width), jnp.float32),
            pltpu.VMEM((RZ, width), jnp.float32),
            pltpu.VMEM_SHARED((N_PAD, width), jnp.float32),
            pltpu.SemaphoreType.DMA((2,)),
            pltpu.SemaphoreType.DMA((2,)),
        ],
    )
    def push(g_hbm, src_hbm, dst_hbm, out_hbm, srcv, dstv, rows, stage, acc,
             gsem, ssem):
        cid = lax.axis_index("c")
        sid = lax.axis_index("s")
        wid = cid * NUM_SUBCORES + sid

        pltpu.sync_copy(src_hbm.at[wid], srcv)
        pltpu.sync_copy(dst_hbm.at[wid], dstv)

        _zero_fill(stage, RZ, width)
        row0 = sid * ROWS_PER_TILE
        for i in range(ROWS_PER_TILE // RZ):
            pltpu.sync_copy(stage, acc.at[pl.ds(row0 + i * RZ, RZ)])
        plsc.subcore_barrier()

        NG = NCHUNK // NB

        def gstart(grp, bank, b):
            j = grp * NB + b
            pltpu.make_async_copy(
                g_hbm.at[srcv.at[pl.ds(j * CHUNK, CHUNK)]],
                rows.at[bank, pl.ds(b * CHUNK, CHUNK)], gsem.at[bank]).start()

        def bank_wait(sem, bank):
            pltpu.make_async_copy(
                g_hbm.at[pl.ds(0, NB * CHUNK)], rows.at[bank],
                sem.at[bank]).wait()

        def sstart(grp, bank, b):
            j = grp * NB + b
            pltpu.make_async_copy(
                rows.at[bank, pl.ds(b * CHUNK, CHUNK)],
                acc.at[dstv.at[j]], ssem.at[bank]).start(add=True)

        for b in range(NB):
            gstart(0, 0, b)

        def one_group(grp, bank):
            bank_wait(gsem, bank)
            for b in range(NB):
                sstart(grp, bank, b)

            @pl.when(grp + 1 < NG)
            def _():
                for b in range(NB):
                    gstart(grp + 1, 1 - bank, b)

            bank_wait(ssem, bank)

        def outer(pp, carry):
            one_group(2 * pp, 0)
            one_group(2 * pp + 1, 1)
            return carry

        lax.fori_loop(0, NG // 2, outer, 0)
        one_group(NG - 1, 0)
        plsc.subcore_barrier()

        for i in range(ROWS_PER_TILE // RZ):
            sl = pl.ds(row0 + i * RZ, RZ)
            pltpu.sync_copy(acc.at[sl], stage)
            if out_width == width:
                pltpu.sync_copy(stage, out_hbm.at[cid, sl])
            else:
                pltpu.sync_copy(stage, out_hbm.at[cid, sl, pl.ds(0, width)])

    return push


_push64 = _make_push(N_HIDDEN, 128)
_push16 = _make_push(N_CLASSES, N_CLASSES)


@functools.partial(
    pl.kernel,
    out_type=jax.ShapeDtypeStruct((NUM_CORES, N_PAD, DEG_W), jnp.float32),
    mesh=_MESH,
    compiler_params=pltpu.CompilerParams(use_tc_tiling_on_sc=False),
    scratch_types=[
        pltpu.VMEM((NCHUNK, CHUNK), jnp.int32),
        pltpu.VMEM((CHUNK, DEG_W), jnp.float32),
        pltpu.VMEM((RZ, DEG_W), jnp.float32),
        pltpu.VMEM_SHARED((N_PAD, DEG_W), jnp.float32),
        pltpu.SemaphoreType.DMA,
    ],
)
def _deg_kernel(dst_hbm, out_hbm, dstv, ones_rows, stage, acc, sem):
    cid = lax.axis_index("c")
    sid = lax.axis_index("s")
    wid = cid * NUM_SUBCORES + sid

    pltpu.sync_copy(dst_hbm.at[wid], dstv)

    one = jnp.ones((16,), jnp.float32)
    for r in range(CHUNK):
        ones_rows[r, pl.ds(0, 16)] = one

    _zero_fill(stage, RZ, DEG_W)
    row0 = sid * ROWS_PER_TILE
    for i in range(ROWS_PER_TILE // RZ):
        pltpu.sync_copy(stage, acc.at[pl.ds(row0 + i * RZ, RZ)])
    plsc.subcore_barrier()

    def sdesc(j):
        return pltpu.make_async_copy(ones_rows, acc.at[dstv.at[j]], sem)

    def fire(j, carry):
        sdesc(j).start(add=True)
        return carry

    lax.fori_loop(0, NCHUNK, fire, 0)

    def drain(j, carry):
        sdesc(j).wait()
        return carry

    lax.fori_loop(0, NCHUNK, drain, 0)
    plsc.subcore_barrier()

    for i in range(ROWS_PER_TILE // RZ):
        sl = pl.ds(row0 + i * RZ, RZ)
        pltpu.sync_copy(acc.at[sl], stage)
        pltpu.sync_copy(stage, out_hbm.at[cid, sl])


def _tc_a_body(psum, x, w1, g1):
    d = lax.rsqrt(psum[:, 0:1] + 1.0)
    g1[...] = d * jnp.dot(x[...], w1[...], preferred_element_type=jnp.float32)


def _tc_a(psum, x, w1):
    return pl.pallas_call(
        _tc_a_body,
        out_shape=jax.ShapeDtypeStruct((N_NODES, N_HIDDEN), jnp.float32),
    )(psum, x, w1)


def _tc_b_body(psum, a_ref, g1, b1, w2, g2):
    d = lax.rsqrt(psum[:, 0:1] + 1.0)
    a = a_ref[...]
    asum = (a[0] + a[1])[:N_NODES, 0:N_HIDDEN] + g1[...]
    z1 = jnp.maximum(d * asum + b1[...], 0.0)
    g2[...] = d * jnp.dot(z1, w2[...], preferred_element_type=jnp.float32)


def _tc_b(psum, a, g1, b1, w2):
    return pl.pallas_call(
        _tc_b_body,
        out_shape=jax.ShapeDtypeStruct((N_NODES, N_CLASSES), jnp.float32),
    )(psum, a, g1, b1, w2)


def _tc_c_body(p_ref, c_ref, g2p, b2t, mblk, out):
    p = p_ref[...]
    d = lax.rsqrt(p[0] + p[1] + 1.0)
    c = c_ref[...]
    z = d * (c[0] + c[1] + g2p[...]) + b2t[...]
    m = jnp.max(z, axis=1, keepdims=True)
    e = jnp.exp(z - m)
    s = jnp.dot(e, mblk[...], preferred_element_type=jnp.float32)
    out[...] = z - m - jnp.log(s)


def _tc_c(p128, c128, g2p, b2t, mblk):
    return pl.pallas_call(
        _tc_c_body,
        out_shape=jax.ShapeDtypeStruct((PK, 128), jnp.float32),
    )(p128, c128, g2p, b2t, mblk)


def kernel(x, edge_index, W1, b1, W2, b2):
    src_full = edge_index[0].reshape(NUM_TILES, EPT)
    dst = edge_index[1].reshape(NUM_TILES, NCHUNK, CHUNK)
    b1r = b1.reshape(1, N_HIDDEN)
    b2t = jnp.tile(b2, 128 // N_CLASSES).reshape(1, 128)
    mblk = jnp.kron(jnp.eye(128 // N_CLASSES, dtype=jnp.float32),
                    jnp.ones((N_CLASSES, N_CLASSES), jnp.float32))

    p = _deg_kernel(dst)
    psum = p[0, :N_NODES] + p[1, :N_NODES]
    g1 = _tc_a(psum, x, W1)
    g1pad = jnp.pad(g1, ((0, N_PAD - N_NODES), (0, 0)))
    a = _push64(g1pad, src_full, dst)
    g2 = _tc_b(psum, a, g1, b1r, W2)
    g2pad = jnp.pad(g2, ((0, N_PAD - N_NODES), (0, 0)))
    c = _push16(g2pad, src_full, dst)
    out128 = _tc_c(p.reshape(NUM_CORES, PK, 128),
                   c.reshape(NUM_CORES, PK, 128),
                   g2pad.reshape(PK, 128), b2t, mblk)
    return out128.reshape(N_PAD, N_CLASSES)[:N_NODES]


# speedup vs baseline: 1.0928x; 1.0928x over previous
import functools

import jax
import jax.numpy as jnp
from jax import lax
from jax.experimental import pallas as pl
from jax.experimental.pallas import tpu as pltpu
from jax.experimental.pallas import tpu_sc as plsc

N_NODES = 10000
N_EDGES = 320000
D_FEAT = 128
N_HIDDEN = 64
N_CLASSES = 16

NUM_CORES = 2
NUM_SUBCORES = 16
NUM_TILES = NUM_CORES * NUM_SUBCORES
N_PAD = 10240
REAL_EPT = N_EDGES // NUM_TILES
SL_PT = N_PAD // NUM_TILES
CHUNK = 80
EPT = REAL_EPT
NCHUNK = EPT // CHUNK
ROWS_PER_TILE = N_PAD // NUM_SUBCORES
RZ = 128
NB = 5
DEG_W = 16
PK = N_PAD * N_CLASSES // 128

_MESH = plsc.VectorSubcoreMesh(core_axis_name="c", subcore_axis_name="s")


def _zero_fill(buf, nrows, width):
    z = jnp.zeros((16,), jnp.float32)
    for r in range(nrows):
        for c in range(width // 16):
            buf[r, pl.ds(c * 16, 16)] = z


def _make_push(width, out_width):
    @functools.partial(
        pl.kernel,
        out_type=jax.ShapeDtypeStruct((NUM_CORES, N_PAD, out_width), jnp.float32),
        mesh=_MESH,
        compiler_params=pltpu.CompilerParams(use_tc_tiling_on_sc=False),
        scratch_types=[
            pltpu.VMEM((EPT,), jnp.int32),
            pltpu.VMEM((NCHUNK, CHUNK), jnp.int32),
            pltpu.VMEM((NB, CHUNK, width), jnp.float32),
            pltpu.VMEM((RZ, width), jnp.float32),
            pltpu.VMEM_SHARED((N_PAD, width), jnp.float32),
            pltpu.SemaphoreType.DMA((NB,)),
        ],
    )
    def push(g_hbm, src_hbm, dst_hbm, out_hbm, srcv, dstv, rows, stage, acc,
             gsem):
        cid = lax.axis_index("c")
        sid = lax.axis_index("s")
        wid = cid * NUM_SUBCORES + sid

        pltpu.sync_copy(src_hbm.at[wid], srcv)
        pltpu.sync_copy(dst_hbm.at[wid], dstv)

        _zero_fill(stage, RZ, width)
        row0 = sid * ROWS_PER_TILE
        for i in range(ROWS_PER_TILE // RZ):
            pltpu.sync_copy(stage, acc.at[pl.ds(row0 + i * RZ, RZ)])
        plsc.subcore_barrier()

        def gather_desc(j, b):
            return pltpu.make_async_copy(
                g_hbm.at[srcv.at[pl.ds(j * CHUNK, CHUNK)]], rows.at[b],
                gsem.at[b])

        for b in range(NB - 1):
            gather_desc(b, b).start()

        def outer(g, carry):
            for b in range(NB):
                j = g * NB + b
                jn = j + NB - 1
                nxt = (b + NB - 1) % NB

                @pl.when(jn < NCHUNK)
                def _():
                    gather_desc(jn, nxt).start()

                gather_desc(j, b).wait()
                pltpu.sync_copy(rows.at[b], acc.at[dstv.at[j]], add=True)
            return carry

        lax.fori_loop(0, NCHUNK // NB, outer, 0)
        plsc.subcore_barrier()

        for i in range(ROWS_PER_TILE // RZ):
            sl = pl.ds(row0 + i * RZ, RZ)
            pltpu.sync_copy(acc.at[sl], stage)
            if out_width == width:
                pltpu.sync_copy(stage, out_hbm.at[cid, sl])
            else:
                pltpu.sync_copy(stage, out_hbm.at[cid, sl, pl.ds(0, width)])

    return push


_push64 = _make_push(N_HIDDEN, 128)
_push16 = _make_push(N_CLASSES, N_CLASSES)


@functools.partial(
    pl.kernel,
    out_type=jax.ShapeDtypeStruct((NUM_CORES, N_PAD, DEG_W), jnp.float32),
    mesh=_MESH,
    compiler_params=pltpu.CompilerParams(use_tc_tiling_on_sc=False),
    scratch_types=[
        pltpu.VMEM((NCHUNK, CHUNK), jnp.int32),
        pltpu.VMEM((CHUNK, DEG_W), jnp.float32),
        pltpu.VMEM((RZ, DEG_W), jnp.float32),
        pltpu.VMEM_SHARED((N_PAD, DEG_W), jnp.float32),
        pltpu.SemaphoreType.DMA,
    ],
)
def _deg_kernel(dst_hbm, out_hbm, dstv, ones_rows, stage, acc, sem):
    cid = lax.axis_index("c")
    sid = lax.axis_index("s")
    wid = cid * NUM_SUBCORES + sid

    pltpu.sync_copy(dst_hbm.at[wid], dstv)

    one = jnp.ones((16,), jnp.float32)
    for r in range(CHUNK):
        ones_rows[r, pl.ds(0, 16)] = one

    _zero_fill(stage, RZ, DEG_W)
    row0 = sid * ROWS_PER_TILE
    for i in range(ROWS_PER_TILE // RZ):
        pltpu.sync_copy(stage, acc.at[pl.ds(row0 + i * RZ, RZ)])
    plsc.subcore_barrier()

    def sdesc(j):
        return pltpu.make_async_copy(ones_rows, acc.at[dstv.at[j]], sem)

    def fire(j, carry):
        sdesc(j).start(add=True)
        return carry

    lax.fori_loop(0, NCHUNK, fire, 0)

    def drain(j, carry):
        sdesc(j).wait()
        return carry

    lax.fori_loop(0, NCHUNK, drain, 0)
    plsc.subcore_barrier()

    for i in range(ROWS_PER_TILE // RZ):
        sl = pl.ds(row0 + i * RZ, RZ)
        pltpu.sync_copy(acc.at[sl], stage)
        pltpu.sync_copy(stage, out_hbm.at[cid, sl])


def _tc_a_body(psum, x, w1, g1):
    d = lax.rsqrt(psum[:, 0:1] + 1.0)
    g1[...] = d * jnp.dot(x[...], w1[...], preferred_element_type=jnp.float32)


def _tc_a(psum, x, w1):
    return pl.pallas_call(
        _tc_a_body,
        out_shape=jax.ShapeDtypeStruct((N_NODES, N_HIDDEN), jnp.float32),
    )(psum, x, w1)


def _tc_b_body(psum, a_ref, g1, b1, w2, g2):
    d = lax.rsqrt(psum[:, 0:1] + 1.0)
    a = a_ref[...]
    asum = (a[0] + a[1])[:N_NODES, 0:N_HIDDEN] + g1[...]
    z1 = jnp.maximum(d * asum + b1[...], 0.0)
    g2[...] = d * jnp.dot(z1, w2[...], preferred_element_type=jnp.float32)


def _tc_b(psum, a, g1, b1, w2):
    return pl.pallas_call(
        _tc_b_body,
        out_shape=jax.ShapeDtypeStruct((N_NODES, N_CLASSES), jnp.float32),
    )(psum, a, g1, b1, w2)


def _tc_c_body(p_ref, c_ref, g2p, b2t, mblk, out):
    p = p_ref[...]
    d = lax.rsqrt(p[0] + p[1] + 1.0)
    c = c_ref[...]
    z = d * (c[0] + c[1] + g2p[...]) + b2t[...]
    m = jnp.max(z, axis=1, keepdims=True)
    e = jnp.exp(z - m)
    s = jnp.dot(e, mblk[...], preferred_element_type=jnp.float32)
    out[...] = z - m - jnp.log(s)


def _tc_c(p128, c128, g2p, b2t, mblk):
    return pl.pallas_call(
        _tc_c_body,
        out_shape=jax.ShapeDtypeStruct((PK, 128), jnp.float32),
    )(p128, c128, g2p, b2t, mblk)


def kernel(x, edge_index, W1, b1, W2, b2):
    src_full = edge_index[0].reshape(NUM_TILES, EPT)
    dst = edge_index[1].reshape(NUM_TILES, NCHUNK, CHUNK)
    b1r = b1.reshape(1, N_HIDDEN)
    b2t = jnp.tile(b2, 128 // N_CLASSES).reshape(1, 128)
    mblk = jnp.kron(jnp.eye(128 // N_CLASSES, dtype=jnp.float32),
                    jnp.ones((N_CLASSES, N_CLASSES), jnp.float32))

    p = _deg_kernel(dst)
    psum = p[0, :N_NODES] + p[1, :N_NODES]
    g1 = _tc_a(psum, x, W1)
    g1pad = jnp.pad(g1, ((0, N_PAD - N_NODES), (0, 0)))
    a = _push64(g1pad, src_full, dst)
    g2 = _tc_b(psum, a, g1, b1r, W2)
    g2pad = jnp.pad(g2, ((0, N_PAD - N_NODES), (0, 0)))
    c = _push16(g2pad, src_full, dst)
    out128 = _tc_c(p.reshape(NUM_CORES, PK, 128),
                   c.reshape(NUM_CORES, PK, 128),
                   g2pad.reshape(PK, 128), b2t, mblk)
    return out128.reshape(N_PAD, N_CLASSES)[:N_NODES]
